# baseline (speedup 1.0000x reference)
import functools
import math

import jax
import jax.numpy as jnp
import numpy as np
from jax import lax
from jax.experimental import pallas as pl
from jax.experimental.pallas import tpu as pltpu
from jax.experimental.pallas import tpu_sc as plsc

_N_NODES = 10000
_N_EDGES = 160000
_MUL = 32
_IN1_LS = [0, 1, 2]
_IN2_LS = [0, 1, 2]
_L_MAX_OUT = 2

_NC = 2
_NS = 16
_LANES = 16
_EPT = _N_EDGES // _NS
_DBLK = 2000
_RANGE = 512
_NPASS = 10


def _su2_cg(j1, j2, j3, m1, m2, m3):
    if m3 != m1 + m2:
        return 0.0
    f = math.factorial
    vmin = int(max(-j1 + j2 + m3, -j1 + m1, 0))
    vmax = int(min(j2 + j3 + m1, j3 - j1 + j2, j3 + m3))
    C = ((2 * j3 + 1) * f(j3 + j1 - j2) * f(j3 - j1 + j2) * f(j1 + j2 - j3) * f(j3 + m3) * f(j3 - m3) / (f(j1 + j2 + j3 + 1) * f(j1 - m1) * f(j1 + m1) * f(j2 - m2) * f(j2 + m2))) ** 0.5
    S = 0.0
    for v in range(vmin, vmax + 1):
        S += (-1.0) ** (v + j2 + m2) / f(v) * f(j2 + j3 + m1 - v) * f(j1 - m1 + v) / (f(j3 - j1 + j2 - v) * f(j3 + m3 - v) * f(v + j1 - j2 - m3))
    return C * S


def _q(l):
    q = np.zeros((2 * l + 1, 2 * l + 1), dtype=np.complex128)
    for m in range(-l, 0):
        q[l + m, l + abs(m)] = 1.0 / 2 ** 0.5
        q[l + m, l - abs(m)] = -1j / 2 ** 0.5
    q[l, l] = 1.0
    for m in range(1, l + 1):
        q[l + m, l + abs(m)] = (-1) ** m / 2 ** 0.5
        q[l + m, l - abs(m)] = 1j * (-1) ** m / 2 ** 0.5
    return (-1j) ** l * q


def _w3j(l1, l2, l3):
    C = np.zeros((2 * l1 + 1, 2 * l2 + 1, 2 * l3 + 1))
    for m1 in range(-l1, l1 + 1):
        for m2 in range(-l2, l2 + 1):
            m3 = m1 + m2
            if abs(m3) <= l3:
                C[l1 + m1, l2 + m2, l3 + m3] = _su2_cg(l1, l2, l3, m1, m2, m3)
    Cr = np.einsum('ij,kl,nm,ikn->jlm', _q(l1), _q(l2), np.conj(_q(l3)), C.astype(np.complex128))
    Cr = np.real(Cr)
    n = np.linalg.norm(Cr)
    return Cr / n if n > 0 else Cr


_PATHS = []
_s1 = [0]
for _l in _IN1_LS:
    _s1.append(_s1[-1] + _MUL * (2 * _l + 1))
_s2 = [0]
for _l in _IN2_LS:
    _s2.append(_s2[-1] + 2 * _l + 1)
_p = 0
_obase = 0
for _i1, _l1 in enumerate(_IN1_LS):
    for _i2, _l2 in enumerate(_IN2_LS):
        for _l3 in range(abs(_l1 - _l2), min(_l1 + _l2, _L_MAX_OUT) + 1):
            C = _w3j(_l1, _l2, _l3)
            knz = {}
            for _i in range(2 * _l1 + 1):
                for _j in range(2 * _l2 + 1):
                    for _k in range(2 * _l3 + 1):
                        v = float(C[_i, _j, _k])
                        if abs(v) > 1e-12:
                            knz.setdefault((_i, _k), []).append((_j, v))
            _PATHS.append(dict(
                p=_p, l1=_l1, l2=_l2, l3=_l3,
                xbase=_s1[_i1], ybase=_s2[_i2], obase=_obase,
                knz=knz,
            ))
            _obase += _MUL * (2 * _l3 + 1)
            _p += 1
_OUT_DIM = _obase
_IN1_DIM = _s1[-1]
_W_DIM = _MUL * len(_PATHS)

_GROUPS = []
for _path in _PATHS:
    nk = len(_path['knz'])
    if (_GROUPS and _GROUPS[-1][0]['xbase'] == _path['xbase']
            and sum(len(q['knz']) for q in _GROUPS[-1]) + nk <= 12):
        _GROUPS[-1].append(_path)
    else:
        _GROUPS.append([_path])


def _splat(v):
    return jnp.zeros((_LANES,), jnp.int32) + v


def _body(in1_h, in2_h, w_h, src_h, dst_h, out_h,
          acc, ids_buf, srcs_buf, dstblk, srcblk, x_buf, w_buf, y_buf,
          out_buf, src_ib, eid_ib, dst_ib, dma_sem):
    c = lax.axis_index("c")
    t = lax.axis_index("s")
    ebase = t * _EPT
    iota = lax.iota(jnp.int32, _LANES)
    zf = jnp.zeros((_LANES,), jnp.float32)

    def zbody(i, _):
        plsc.store_scatter(out_buf, [iota, _splat(i)], zf)
        return 0

    def one_pass(r, _):
        base = (2 * r + c) * _RANGE
        nrows = jnp.minimum(_RANGE, _N_NODES - base)

        lax.fori_loop(0, _OUT_DIM, zbody, 0)
        for q in range(_RANGE // (16 * _NS)):
            cidx = t + _NS * q

            @pl.when(cidx * 16 < nrows)
            def _():
                pltpu.sync_copy(out_buf, acc.at[pl.ds(cidx * 16, 16)])

        @pl.when(t == _NS - 1)
        def _():
            pltpu.sync_copy(out_buf.at[pl.ds(0, 1)], acc.at[pl.ds(_RANGE, 1)])

        plsc.subcore_barrier()

        def cblock(b, cnt):
            da = pltpu.async_copy(
                dst_h.at[pl.ds(ebase + b * _DBLK, _DBLK)], dstblk, dma_sem)
            db = pltpu.async_copy(
                src_h.at[pl.ds(ebase + b * _DBLK, _DBLK)], srcblk, dma_sem)
            da.wait()
            db.wait()

            def cbody(i, cnt):
                d = dstblk[pl.ds(i * 16, 16)]
                s = srcblk[pl.ds(i * 16, 16)]
                m = (d >= base) & (d < base + _RANGE)
                pos = plsc.cumsum(jnp.where(m, 1, 0)) + cnt - 1
                packed = (b * _DBLK + i * 16 + iota) * 1024 + (d - base)
                plsc.store_scatter(ids_buf, [pos], packed, mask=m)
                plsc.store_scatter(srcs_buf, [pos], s, mask=m)
                return cnt + plsc.all_reduce_population_count(m)

            return lax.fori_loop(0, _DBLK // 16, cbody, cnt)

        cntv = lax.fori_loop(0, _EPT // _DBLK, cblock,
                             jnp.zeros((_LANES,), jnp.int32))
        plsc.store_scatter(ids_buf, [cntv + iota], _splat(0))
        plsc.store_scatter(srcs_buf, [cntv + iota], _splat(0))
        cnt = jnp.max(cntv)
        nch = (cnt + 15) // 16

        def chbody(ch, _):
            vv = ids_buf[pl.ds(ch * 16, 16)]
            validv = (ch * 16 + iota) < cntv
            ids_loc = lax.shift_right_logical(vv, 10)
            dstoff = vv & 1023
            eid_ib[...] = ids_loc + ebase
            dst_ib[...] = jnp.where(validv, dstoff, _RANGE)
            src_ib[...] = srcs_buf[pl.ds(ch * 16, 16)]
            dx = pltpu.async_copy(in1_h.at[src_ib], x_buf, dma_sem)
            dw = pltpu.async_copy(w_h.at[eid_ib], w_buf, dma_sem)
            dy = pltpu.async_copy(in2_h.at[eid_ib], y_buf, dma_sem)
            dx.wait()
            dw.wait()
            dy.wait()

            ys = [plsc.load_gather(y_buf, [iota, _splat(j)]) for j in range(_s2[-1])]

            for group in _GROUPS:
                st1 = 2 * group[0]['l1'] + 1
                xbase = group[0]['xbase']
                Ks = []
                for path in group:
                    K = {}
                    for (i, k), terms in path['knz'].items():
                        acc_v = None
                        for (j, coef) in terms:
                            term = ys[path['ybase'] + j] * coef
                            acc_v = term if acc_v is None else acc_v + term
                        K[(i, k)] = acc_v
                    Ks.append(K)
                i_used = sorted({i for K in Ks for (i, _k) in K})

                def ubody(u, group=group, Ks=Ks, i_used=i_used,
                          st1=st1, xbase=xbase):
                    xs = {
                        i: plsc.load_gather(
                            x_buf, [iota, _splat(xbase + u * st1 + i)])
                        for i in i_used
                    }
                    for path, K in zip(group, Ks):
                        st3 = 2 * path['l3'] + 1
                        wv = plsc.load_gather(
                            w_buf, [iota, _splat(path['p'] * _MUL + u)])
                        for k in range(st3):
                            acc_k = None
                            for i in i_used:
                                if (i, k) in K:
                                    term = K[(i, k)] * xs[i]
                                    acc_k = (term if acc_k is None
                                             else acc_k + term)
                            if acc_k is None:
                                continue
                            plsc.store_scatter(
                                out_buf,
                                [iota, _splat(path['obase'] + u * st3 + k)],
                                acc_k * wv)

                plsc.parallel_loop(0, _MUL, 1, unroll=2)(ubody)

            pltpu.sync_copy(out_buf, acc.at[dst_ib], add=True)
            return 0

        lax.fori_loop(0, nch, chbody, 0)
        plsc.subcore_barrier()

        for q in range(_RANGE // (16 * _NS)):
            cidx = t + _NS * q

            @pl.when(cidx * 16 < nrows)
            def _():
                pltpu.sync_copy(acc.at[pl.ds(cidx * 16, 16)],
                                out_h.at[pl.ds(base + cidx * 16, 16)])

        plsc.subcore_barrier()
        return 0

    lax.fori_loop(0, _NPASS, one_pass, 0)


def kernel(in1, in2, weight, per_edge_src, per_edge_dst):
    in2p = jnp.pad(in2, ((0, 0), (0, 16 - _s2[-1])))
    mesh = plsc.VectorSubcoreMesh(
        core_axis_name="c", subcore_axis_name="s",
        num_cores=_NC, num_subcores=_NS)
    f = pl.kernel(
        _body,
        out_type=jax.ShapeDtypeStruct((_N_NODES, _OUT_DIM), jnp.float32),
        mesh=mesh,
        compiler_params=pltpu.CompilerParams(
            use_tc_tiling_on_sc=False, needs_layout_passes=False),
        scratch_types=[
            pltpu.VMEM_SHARED((_RANGE + 1, _OUT_DIM), jnp.float32),
            pltpu.VMEM((_EPT + 16,), jnp.int32),
            pltpu.VMEM((_EPT + 16,), jnp.int32),
            pltpu.VMEM((_DBLK,), jnp.int32),
            pltpu.VMEM((_DBLK,), jnp.int32),
            pltpu.VMEM((16, _IN1_DIM), jnp.float32),
            pltpu.VMEM((16, _W_DIM), jnp.float32),
            pltpu.VMEM((16, 16), jnp.float32),
            pltpu.VMEM((16, _OUT_DIM), jnp.float32),
            pltpu.VMEM((16,), jnp.int32),
            pltpu.VMEM((16,), jnp.int32),
            pltpu.VMEM((16,), jnp.int32),
            pltpu.SemaphoreType.DMA,
        ],
    )
    return f(in1, in2p, weight, per_edge_src, per_edge_dst)


# speedup vs baseline: 1.1324x; 1.1324x over previous
import functools
import math

import jax
import jax.numpy as jnp
import numpy as np
from jax import lax
from jax.experimental import pallas as pl
from jax.experimental.pallas import tpu as pltpu
from jax.experimental.pallas import tpu_sc as plsc

_N_NODES = 10000
_N_EDGES = 160000
_MUL = 32
_IN1_LS = [0, 1, 2]
_IN2_LS = [0, 1, 2]
_L_MAX_OUT = 2

_NC = 2
_NS = 16
_LANES = 16
_EPT = _N_EDGES // _NS
_DBLK = 2000
_RANGE = 512
_NPASS = 10


def _su2_cg(j1, j2, j3, m1, m2, m3):
    if m3 != m1 + m2:
        return 0.0
    f = math.factorial
    vmin = int(max(-j1 + j2 + m3, -j1 + m1, 0))
    vmax = int(min(j2 + j3 + m1, j3 - j1 + j2, j3 + m3))
    C = ((2 * j3 + 1) * f(j3 + j1 - j2) * f(j3 - j1 + j2) * f(j1 + j2 - j3) * f(j3 + m3) * f(j3 - m3) / (f(j1 + j2 + j3 + 1) * f(j1 - m1) * f(j1 + m1) * f(j2 - m2) * f(j2 + m2))) ** 0.5
    S = 0.0
    for v in range(vmin, vmax + 1):
        S += (-1.0) ** (v + j2 + m2) / f(v) * f(j2 + j3 + m1 - v) * f(j1 - m1 + v) / (f(j3 - j1 + j2 - v) * f(j3 + m3 - v) * f(v + j1 - j2 - m3))
    return C * S


def _q(l):
    q = np.zeros((2 * l + 1, 2 * l + 1), dtype=np.complex128)
    for m in range(-l, 0):
        q[l + m, l + abs(m)] = 1.0 / 2 ** 0.5
        q[l + m, l - abs(m)] = -1j / 2 ** 0.5
    q[l, l] = 1.0
    for m in range(1, l + 1):
        q[l + m, l + abs(m)] = (-1) ** m / 2 ** 0.5
        q[l + m, l - abs(m)] = 1j * (-1) ** m / 2 ** 0.5
    return (-1j) ** l * q


def _w3j(l1, l2, l3):
    C = np.zeros((2 * l1 + 1, 2 * l2 + 1, 2 * l3 + 1))
    for m1 in range(-l1, l1 + 1):
        for m2 in range(-l2, l2 + 1):
            m3 = m1 + m2
            if abs(m3) <= l3:
                C[l1 + m1, l2 + m2, l3 + m3] = _su2_cg(l1, l2, l3, m1, m2, m3)
    Cr = np.einsum('ij,kl,nm,ikn->jlm', _q(l1), _q(l2), np.conj(_q(l3)), C.astype(np.complex128))
    Cr = np.real(Cr)
    n = np.linalg.norm(Cr)
    return Cr / n if n > 0 else Cr


_PATHS = []
_s1 = [0]
for _l in _IN1_LS:
    _s1.append(_s1[-1] + _MUL * (2 * _l + 1))
_s2 = [0]
for _l in _IN2_LS:
    _s2.append(_s2[-1] + 2 * _l + 1)
_p = 0
_obase = 0
for _i1, _l1 in enumerate(_IN1_LS):
    for _i2, _l2 in enumerate(_IN2_LS):
        for _l3 in range(abs(_l1 - _l2), min(_l1 + _l2, _L_MAX_OUT) + 1):
            C = _w3j(_l1, _l2, _l3)
            knz = {}
            for _i in range(2 * _l1 + 1):
                for _j in range(2 * _l2 + 1):
                    for _k in range(2 * _l3 + 1):
                        v = float(C[_i, _j, _k])
                        if abs(v) > 1e-12:
                            knz.setdefault((_i, _k), []).append((_j, v))
            _PATHS.append(dict(
                p=_p, l1=_l1, l2=_l2, l3=_l3,
                xbase=_s1[_i1], ybase=_s2[_i2], obase=_obase,
                knz=knz,
            ))
            _obase += _MUL * (2 * _l3 + 1)
            _p += 1
_OUT_DIM = _obase
_IN1_DIM = _s1[-1]
_W_DIM = _MUL * len(_PATHS)

_GROUPS = []
for _path in _PATHS:
    nk = len(_path['knz'])
    if (_GROUPS and _GROUPS[-1][0]['xbase'] == _path['xbase']
            and sum(len(q['knz']) for q in _GROUPS[-1]) + nk <= 999):
        _GROUPS[-1].append(_path)
    else:
        _GROUPS.append([_path])


def _splat(v):
    return jnp.zeros((_LANES,), jnp.int32) + v


def _body(in1_h, in2_h, w_h, src_h, dst_h, out_h,
          acc, ids_buf, srcs_buf, dstblk, srcblk, x_buf, w_buf, y_buf,
          out_buf, src_ib, eid_ib, dst_ib, dma_sem):
    c = lax.axis_index("c")
    t = lax.axis_index("s")
    ebase = t * _EPT
    iota = lax.iota(jnp.int32, _LANES)
    zf = jnp.zeros((_LANES,), jnp.float32)

    def zbody(i, _):
        plsc.store_scatter(out_buf, [iota, _splat(i)], zf)
        return 0

    def one_pass(r, _):
        base = (2 * r + c) * _RANGE
        nrows = jnp.minimum(_RANGE, _N_NODES - base)

        lax.fori_loop(0, _OUT_DIM, zbody, 0)
        for q in range(_RANGE // (16 * _NS)):
            cidx = t + _NS * q

            @pl.when(cidx * 16 < nrows)
            def _():
                pltpu.sync_copy(out_buf, acc.at[pl.ds(cidx * 16, 16)])

        @pl.when(t == _NS - 1)
        def _():
            pltpu.sync_copy(out_buf.at[pl.ds(0, 1)], acc.at[pl.ds(_RANGE, 1)])

        plsc.subcore_barrier()

        def cblock(b, cnt):
            da = pltpu.async_copy(
                dst_h.at[pl.ds(ebase + b * _DBLK, _DBLK)], dstblk, dma_sem)
            db = pltpu.async_copy(
                src_h.at[pl.ds(ebase + b * _DBLK, _DBLK)], srcblk, dma_sem)
            da.wait()
            db.wait()

            def cbody(i, cnt):
                d = dstblk[pl.ds(i * 16, 16)]
                s = srcblk[pl.ds(i * 16, 16)]
                m = (d >= base) & (d < base + _RANGE)
                pos = plsc.cumsum(jnp.where(m, 1, 0)) + cnt - 1
                packed = (b * _DBLK + i * 16 + iota) * 1024 + (d - base)
                plsc.store_scatter(ids_buf, [pos], packed, mask=m)
                plsc.store_scatter(srcs_buf, [pos], s, mask=m)
                return cnt + plsc.all_reduce_population_count(m)

            return lax.fori_loop(0, _DBLK // 16, cbody, cnt)

        cntv = lax.fori_loop(0, _EPT // _DBLK, cblock,
                             jnp.zeros((_LANES,), jnp.int32))
        plsc.store_scatter(ids_buf, [cntv + iota], _splat(0))
        plsc.store_scatter(srcs_buf, [cntv + iota], _splat(0))
        cnt = jnp.max(cntv)
        nch = (cnt + 15) // 16

        def chbody(ch, _):
            vv = ids_buf[pl.ds(ch * 16, 16)]
            validv = (ch * 16 + iota) < cntv
            ids_loc = lax.shift_right_logical(vv, 10)
            dstoff = vv & 1023
            eid_ib[...] = ids_loc + ebase
            dst_ib[...] = jnp.where(validv, dstoff, _RANGE)
            src_ib[...] = srcs_buf[pl.ds(ch * 16, 16)]
            dx = pltpu.async_copy(in1_h.at[src_ib], x_buf, dma_sem)
            dw = pltpu.async_copy(w_h.at[eid_ib], w_buf, dma_sem)
            dy = pltpu.async_copy(in2_h.at[eid_ib], y_buf, dma_sem)
            dx.wait()
            dw.wait()
            dy.wait()

            ys = [plsc.load_gather(y_buf, [iota, _splat(j)]) for j in range(_s2[-1])]

            for group in _GROUPS:
                st1 = 2 * group[0]['l1'] + 1
                xbase = group[0]['xbase']
                Ks = []
                for path in group:
                    K = {}
                    for (i, k), terms in path['knz'].items():
                        acc_v = None
                        for (j, coef) in terms:
                            term = ys[path['ybase'] + j] * coef
                            acc_v = term if acc_v is None else acc_v + term
                        K[(i, k)] = acc_v
                    Ks.append(K)
                i_used = sorted({i for K in Ks for (i, _k) in K})

                def ubody(u, group=group, Ks=Ks, i_used=i_used,
                          st1=st1, xbase=xbase):
                    xs = {
                        i: plsc.load_gather(
                            x_buf, [iota, _splat(xbase + u * st1 + i)])
                        for i in i_used
                    }
                    for path, K in zip(group, Ks):
                        st3 = 2 * path['l3'] + 1
                        wv = plsc.load_gather(
                            w_buf, [iota, _splat(path['p'] * _MUL + u)])
                        for k in range(st3):
                            acc_k = None
                            for i in i_used:
                                if (i, k) in K:
                                    term = K[(i, k)] * xs[i]
                                    acc_k = (term if acc_k is None
                                             else acc_k + term)
                            if acc_k is None:
                                continue
                            plsc.store_scatter(
                                out_buf,
                                [iota, _splat(path['obase'] + u * st3 + k)],
                                acc_k * wv)

                plsc.parallel_loop(0, _MUL, 1, unroll=2)(ubody)

            pltpu.sync_copy(out_buf, acc.at[dst_ib], add=True)
            return 0

        lax.fori_loop(0, nch, chbody, 0)
        plsc.subcore_barrier()

        for q in range(_RANGE // (16 * _NS)):
            cidx = t + _NS * q

            @pl.when(cidx * 16 < nrows)
            def _():
                pltpu.sync_copy(acc.at[pl.ds(cidx * 16, 16)],
                                out_h.at[pl.ds(base + cidx * 16, 16)])

        plsc.subcore_barrier()
        return 0

    lax.fori_loop(0, _NPASS, one_pass, 0)


def kernel(in1, in2, weight, per_edge_src, per_edge_dst):
    in2p = jnp.pad(in2, ((0, 0), (0, 16 - _s2[-1])))
    mesh = plsc.VectorSubcoreMesh(
        core_axis_name="c", subcore_axis_name="s",
        num_cores=_NC, num_subcores=_NS)
    f = pl.kernel(
        _body,
        out_type=jax.ShapeDtypeStruct((_N_NODES, _OUT_DIM), jnp.float32),
        mesh=mesh,
        compiler_params=pltpu.CompilerParams(
            use_tc_tiling_on_sc=False, needs_layout_passes=False),
        scratch_types=[
            pltpu.VMEM_SHARED((_RANGE + 1, _OUT_DIM), jnp.float32),
            pltpu.VMEM((_EPT + 16,), jnp.int32),
            pltpu.VMEM((_EPT + 16,), jnp.int32),
            pltpu.VMEM((_DBLK,), jnp.int32),
            pltpu.VMEM((_DBLK,), jnp.int32),
            pltpu.VMEM((16, _IN1_DIM), jnp.float32),
            pltpu.VMEM((16, _W_DIM), jnp.float32),
            pltpu.VMEM((16, 16), jnp.float32),
            pltpu.VMEM((16, _OUT_DIM), jnp.float32),
            pltpu.VMEM((16,), jnp.int32),
            pltpu.VMEM((16,), jnp.int32),
            pltpu.VMEM((16,), jnp.int32),
            pltpu.SemaphoreType.DMA,
        ],
    )
    return f(in1, in2p, weight, per_edge_src, per_edge_dst)


# speedup vs baseline: 1.1412x; 1.0078x over previous
import functools
import math

import jax
import jax.numpy as jnp
import numpy as np
from jax import lax
from jax.experimental import pallas as pl
from jax.experimental.pallas import tpu as pltpu
from jax.experimental.pallas import tpu_sc as plsc

_N_NODES = 10000
_N_EDGES = 160000
_MUL = 32
_IN1_LS = [0, 1, 2]
_IN2_LS = [0, 1, 2]
_L_MAX_OUT = 2

_NC = 2
_NS = 16
_LANES = 16
_EPT = _N_EDGES // _NS
_DBLK = 2000
_RANGE = 512
_NPASS = 10


def _su2_cg(j1, j2, j3, m1, m2, m3):
    if m3 != m1 + m2:
        return 0.0
    f = math.factorial
    vmin = int(max(-j1 + j2 + m3, -j1 + m1, 0))
    vmax = int(min(j2 + j3 + m1, j3 - j1 + j2, j3 + m3))
    C = ((2 * j3 + 1) * f(j3 + j1 - j2) * f(j3 - j1 + j2) * f(j1 + j2 - j3) * f(j3 + m3) * f(j3 - m3) / (f(j1 + j2 + j3 + 1) * f(j1 - m1) * f(j1 + m1) * f(j2 - m2) * f(j2 + m2))) ** 0.5
    S = 0.0
    for v in range(vmin, vmax + 1):
        S += (-1.0) ** (v + j2 + m2) / f(v) * f(j2 + j3 + m1 - v) * f(j1 - m1 + v) / (f(j3 - j1 + j2 - v) * f(j3 + m3 - v) * f(v + j1 - j2 - m3))
    return C * S


def _q(l):
    q = np.zeros((2 * l + 1, 2 * l + 1), dtype=np.complex128)
    for m in range(-l, 0):
        q[l + m, l + abs(m)] = 1.0 / 2 ** 0.5
        q[l + m, l - abs(m)] = -1j / 2 ** 0.5
    q[l, l] = 1.0
    for m in range(1, l + 1):
        q[l + m, l + abs(m)] = (-1) ** m / 2 ** 0.5
        q[l + m, l - abs(m)] = 1j * (-1) ** m / 2 ** 0.5
    return (-1j) ** l * q


def _w3j(l1, l2, l3):
    C = np.zeros((2 * l1 + 1, 2 * l2 + 1, 2 * l3 + 1))
    for m1 in range(-l1, l1 + 1):
        for m2 in range(-l2, l2 + 1):
            m3 = m1 + m2
            if abs(m3) <= l3:
                C[l1 + m1, l2 + m2, l3 + m3] = _su2_cg(l1, l2, l3, m1, m2, m3)
    Cr = np.einsum('ij,kl,nm,ikn->jlm', _q(l1), _q(l2), np.conj(_q(l3)), C.astype(np.complex128))
    Cr = np.real(Cr)
    n = np.linalg.norm(Cr)
    return Cr / n if n > 0 else Cr


_PATHS = []
_s1 = [0]
for _l in _IN1_LS:
    _s1.append(_s1[-1] + _MUL * (2 * _l + 1))
_s2 = [0]
for _l in _IN2_LS:
    _s2.append(_s2[-1] + 2 * _l + 1)
_p = 0
_obase = 0
for _i1, _l1 in enumerate(_IN1_LS):
    for _i2, _l2 in enumerate(_IN2_LS):
        for _l3 in range(abs(_l1 - _l2), min(_l1 + _l2, _L_MAX_OUT) + 1):
            C = _w3j(_l1, _l2, _l3)
            knz = {}
            for _i in range(2 * _l1 + 1):
                for _j in range(2 * _l2 + 1):
                    for _k in range(2 * _l3 + 1):
                        v = float(C[_i, _j, _k])
                        if abs(v) > 1e-12:
                            knz.setdefault((_i, _k), []).append((_j, v))
            _PATHS.append(dict(
                p=_p, l1=_l1, l2=_l2, l3=_l3,
                xbase=_s1[_i1], ybase=_s2[_i2], obase=_obase,
                knz=knz,
            ))
            _obase += _MUL * (2 * _l3 + 1)
            _p += 1
_OUT_DIM = _obase
_IN1_DIM = _s1[-1]
_W_DIM = _MUL * len(_PATHS)

_GROUPS = []
for _path in _PATHS:
    nk = len(_path['knz'])
    if (_GROUPS and _GROUPS[-1][0]['xbase'] == _path['xbase']
            and sum(len(q['knz']) for q in _GROUPS[-1]) + nk <= 999):
        _GROUPS[-1].append(_path)
    else:
        _GROUPS.append([_path])


def _splat(v):
    return jnp.zeros((_LANES,), jnp.int32) + v


def _body(in1_h, in2_h, w_h, src_h, dst_h, out_h,
          acc, ids_buf, srcs_buf, dstblk, srcblk, x_buf, w_buf, y_buf,
          out_buf, src_ib, eid_ib, dst_ib, dma_sem):
    c = lax.axis_index("c")
    t = lax.axis_index("s")
    ebase = t * _EPT
    iota = lax.iota(jnp.int32, _LANES)
    zf = jnp.zeros((_LANES,), jnp.float32)

    def zbody(i, _):
        plsc.store_scatter(out_buf, [iota, _splat(i)], zf)
        return 0

    def one_pass(r, _):
        base = (2 * r + c) * _RANGE
        nrows = jnp.minimum(_RANGE, _N_NODES - base)

        lax.fori_loop(0, _OUT_DIM, zbody, 0)
        for q in range(_RANGE // (16 * _NS)):
            cidx = t + _NS * q

            @pl.when(cidx * 16 < nrows)
            def _():
                pltpu.sync_copy(out_buf, acc.at[pl.ds(cidx * 16, 16)])

        @pl.when(t == _NS - 1)
        def _():
            pltpu.sync_copy(out_buf.at[pl.ds(0, 1)], acc.at[pl.ds(_RANGE, 1)])

        plsc.subcore_barrier()

        def cblock(b, cnt):
            da = pltpu.async_copy(
                dst_h.at[pl.ds(ebase + b * _DBLK, _DBLK)], dstblk, dma_sem)
            db = pltpu.async_copy(
                src_h.at[pl.ds(ebase + b * _DBLK, _DBLK)], srcblk, dma_sem)
            da.wait()
            db.wait()

            def cbody(i, cnt):
                d = dstblk[pl.ds(i * 16, 16)]
                s = srcblk[pl.ds(i * 16, 16)]
                m = (d >= base) & (d < base + _RANGE)
                pos = plsc.cumsum(jnp.where(m, 1, 0)) + cnt - 1
                packed = (b * _DBLK + i * 16 + iota) * 1024 + (d - base)
                plsc.store_scatter(ids_buf, [pos], packed, mask=m)
                plsc.store_scatter(srcs_buf, [pos], s, mask=m)
                return cnt + plsc.all_reduce_population_count(m)

            return lax.fori_loop(0, _DBLK // 16, cbody, cnt)

        cntv = lax.fori_loop(0, _EPT // _DBLK, cblock,
                             jnp.zeros((_LANES,), jnp.int32))
        plsc.store_scatter(ids_buf, [cntv + iota], _splat(0))
        plsc.store_scatter(srcs_buf, [cntv + iota], _splat(0))
        cnt = jnp.max(cntv)
        nch = (cnt + 15) // 16

        def chbody(ch, _):
            vv = ids_buf[pl.ds(ch * 16, 16)]
            validv = (ch * 16 + iota) < cntv
            ids_loc = lax.shift_right_logical(vv, 10)
            dstoff = vv & 1023
            eid_ib[...] = ids_loc + ebase
            dst_ib[...] = jnp.where(validv, dstoff, _RANGE)
            src_ib[...] = srcs_buf[pl.ds(ch * 16, 16)]
            dx = pltpu.async_copy(in1_h.at[src_ib], x_buf, dma_sem)
            dw = pltpu.async_copy(w_h.at[eid_ib], w_buf, dma_sem)
            dy = pltpu.async_copy(in2_h.at[eid_ib], y_buf, dma_sem)
            dx.wait()
            dw.wait()
            dy.wait()

            ys = [plsc.load_gather(y_buf, [iota, _splat(j)]) for j in range(_s2[-1])]

            for group in _GROUPS:
                st1 = 2 * group[0]['l1'] + 1
                xbase = group[0]['xbase']
                Ks = []
                for path in group:
                    K = {}
                    for (i, k), terms in path['knz'].items():
                        acc_v = None
                        for (j, coef) in terms:
                            term = ys[path['ybase'] + j] * coef
                            acc_v = term if acc_v is None else acc_v + term
                        K[(i, k)] = acc_v
                    Ks.append(K)
                i_used = sorted({i for K in Ks for (i, _k) in K})

                one_c = _splat(1)
                st1_c = _splat(st1)
                st3_cs = [_splat(2 * path['l3'] + 1) for path in group]
                carry0 = (
                    tuple(_splat(xbase + i) for i in i_used),
                    tuple(_splat(path['p'] * _MUL) for path in group),
                    tuple(_splat(path['obase']) for path in group),
                )

                def ubody(u, carry, group=group, Ks=Ks, i_used=i_used):
                    xcols, wcols, ocols = carry
                    xs = {
                        i: plsc.load_gather(x_buf, [iota, xcols[n]])
                        for n, i in enumerate(i_used)
                    }
                    for pi, (path, K) in enumerate(zip(group, Ks)):
                        st3 = 2 * path['l3'] + 1
                        wv = plsc.load_gather(w_buf, [iota, wcols[pi]])
                        idx = ocols[pi]
                        for k in range(st3):
                            acc_k = None
                            for i in i_used:
                                if (i, k) in K:
                                    term = K[(i, k)] * xs[i]
                                    acc_k = (term if acc_k is None
                                             else acc_k + term)
                            if acc_k is not None:
                                plsc.store_scatter(
                                    out_buf, [iota, idx], acc_k * wv)
                            if k + 1 < st3:
                                idx = idx + one_c
                    return (
                        tuple(xc + st1_c for xc in xcols),
                        tuple(wc + one_c for wc in wcols),
                        tuple(oc + st3_cs[pi] for pi, oc in enumerate(ocols)),
                    )

                plsc.parallel_loop(0, _MUL, 1, unroll=2, carry=carry0)(ubody)

            pltpu.sync_copy(out_buf, acc.at[dst_ib], add=True)
            return 0

        lax.fori_loop(0, nch, chbody, 0)
        plsc.subcore_barrier()

        for q in range(_RANGE // (16 * _NS)):
            cidx = t + _NS * q

            @pl.when(cidx * 16 < nrows)
            def _():
                pltpu.sync_copy(acc.at[pl.ds(cidx * 16, 16)],
                                out_h.at[pl.ds(base + cidx * 16, 16)])

        plsc.subcore_barrier()
        return 0

    lax.fori_loop(0, _NPASS, one_pass, 0)


def kernel(in1, in2, weight, per_edge_src, per_edge_dst):
    in2p = jnp.pad(in2, ((0, 0), (0, 16 - _s2[-1])))
    mesh = plsc.VectorSubcoreMesh(
        core_axis_name="c", subcore_axis_name="s",
        num_cores=_NC, num_subcores=_NS)
    f = pl.kernel(
        _body,
        out_type=jax.ShapeDtypeStruct((_N_NODES, _OUT_DIM), jnp.float32),
        mesh=mesh,
        compiler_params=pltpu.CompilerParams(
            use_tc_tiling_on_sc=False, needs_layout_passes=False),
        scratch_types=[
            pltpu.VMEM_SHARED((_RANGE + 1, _OUT_DIM), jnp.float32),
            pltpu.VMEM((_EPT + 16,), jnp.int32),
            pltpu.VMEM((_EPT + 16,), jnp.int32),
            pltpu.VMEM((_DBLK,), jnp.int32),
            pltpu.VMEM((_DBLK,), jnp.int32),
            pltpu.VMEM((16, _IN1_DIM), jnp.float32),
            pltpu.VMEM((16, _W_DIM), jnp.float32),
            pltpu.VMEM((16, 16), jnp.float32),
            pltpu.VMEM((16, _OUT_DIM), jnp.float32),
            pltpu.VMEM((16,), jnp.int32),
            pltpu.VMEM((16,), jnp.int32),
            pltpu.VMEM((16,), jnp.int32),
            pltpu.SemaphoreType.DMA,
        ],
    )
    return f(in1, in2p, weight, per_edge_src, per_edge_dst)


# speedup vs baseline: 2.1108x; 1.8497x over previous
import functools
import math

import jax
import jax.numpy as jnp
import numpy as np
from jax import lax
from jax.experimental import pallas as pl
from jax.experimental.pallas import tpu as pltpu
from jax.experimental.pallas import tpu_sc as plsc

_N_NODES = 10000
_N_EDGES = 160000
_MUL = 32
_IN1_LS = [0, 1, 2]
_IN2_LS = [0, 1, 2]
_L_MAX_OUT = 2

_NC = 2
_NS = 16
_LANES = 16
_EPT = _N_EDGES // _NS
_DBLK = 2000
_RANGE = 256
_NPASS = 20


def _su2_cg(j1, j2, j3, m1, m2, m3):
    if m3 != m1 + m2:
        return 0.0
    f = math.factorial
    vmin = int(max(-j1 + j2 + m3, -j1 + m1, 0))
    vmax = int(min(j2 + j3 + m1, j3 - j1 + j2, j3 + m3))
    C = ((2 * j3 + 1) * f(j3 + j1 - j2) * f(j3 - j1 + j2) * f(j1 + j2 - j3) * f(j3 + m3) * f(j3 - m3) / (f(j1 + j2 + j3 + 1) * f(j1 - m1) * f(j1 + m1) * f(j2 - m2) * f(j2 + m2))) ** 0.5
    S = 0.0
    for v in range(vmin, vmax + 1):
        S += (-1.0) ** (v + j2 + m2) / f(v) * f(j2 + j3 + m1 - v) * f(j1 - m1 + v) / (f(j3 - j1 + j2 - v) * f(j3 + m3 - v) * f(v + j1 - j2 - m3))
    return C * S


def _q(l):
    q = np.zeros((2 * l + 1, 2 * l + 1), dtype=np.complex128)
    for m in range(-l, 0):
        q[l + m, l + abs(m)] = 1.0 / 2 ** 0.5
        q[l + m, l - abs(m)] = -1j / 2 ** 0.5
    q[l, l] = 1.0
    for m in range(1, l + 1):
        q[l + m, l + abs(m)] = (-1) ** m / 2 ** 0.5
        q[l + m, l - abs(m)] = 1j * (-1) ** m / 2 ** 0.5
    return (-1j) ** l * q


def _w3j(l1, l2, l3):
    C = np.zeros((2 * l1 + 1, 2 * l2 + 1, 2 * l3 + 1))
    for m1 in range(-l1, l1 + 1):
        for m2 in range(-l2, l2 + 1):
            m3 = m1 + m2
            if abs(m3) <= l3:
                C[l1 + m1, l2 + m2, l3 + m3] = _su2_cg(l1, l2, l3, m1, m2, m3)
    Cr = np.einsum('ij,kl,nm,ikn->jlm', _q(l1), _q(l2), np.conj(_q(l3)), C.astype(np.complex128))
    Cr = np.real(Cr)
    n = np.linalg.norm(Cr)
    return Cr / n if n > 0 else Cr


_PATHS = []
_s1 = [0]
for _l in _IN1_LS:
    _s1.append(_s1[-1] + _MUL * (2 * _l + 1))
_s2 = [0]
for _l in _IN2_LS:
    _s2.append(_s2[-1] + 2 * _l + 1)
_p = 0
_obase = 0
for _i1, _l1 in enumerate(_IN1_LS):
    for _i2, _l2 in enumerate(_IN2_LS):
        for _l3 in range(abs(_l1 - _l2), min(_l1 + _l2, _L_MAX_OUT) + 1):
            C = _w3j(_l1, _l2, _l3)
            knz = {}
            for _i in range(2 * _l1 + 1):
                for _j in range(2 * _l2 + 1):
                    for _k in range(2 * _l3 + 1):
                        v = float(C[_i, _j, _k])
                        if abs(v) > 1e-12:
                            knz.setdefault((_i, _k), []).append((_j, v))
            _PATHS.append(dict(
                p=_p, l1=_l1, l2=_l2, l3=_l3,
                xbase=_s1[_i1], ybase=_s2[_i2], obase=_obase,
                knz=knz,
            ))
            _obase += _MUL * (2 * _l3 + 1)
            _p += 1
_OUT_DIM = _obase
_IN1_DIM = _s1[-1]
_W_DIM = _MUL * len(_PATHS)

_GROUPS = []
for _path in _PATHS:
    nk = len(_path['knz'])
    if (_GROUPS and _GROUPS[-1][0]['xbase'] == _path['xbase']
            and sum(len(q['knz']) for q in _GROUPS[-1]) + nk <= 999):
        _GROUPS[-1].append(_path)
    else:
        _GROUPS.append([_path])


def _splat(v):
    return jnp.zeros((_LANES,), jnp.int32) + v


def _body(in1_h, in2_h, w_h, src_h, dst_h, out_h,
          acc, ids_buf, srcs_buf, dstblk, srcblk, x_stage, x_buf, w_stage,
          w_buf, y_stage, y_buf, out_buf, out_stage, src_ib, eid_ib, dst_ib,
          dma_sem):
    c = lax.axis_index("c")
    t = lax.axis_index("s")
    ebase = t * _EPT
    iota = lax.iota(jnp.int32, _LANES)
    zf = jnp.zeros((_LANES,), jnp.float32)

    def zstage(cb, _):
        for e in range(16):
            out_stage[e, pl.ds(cb * 16, 16)] = zf
        return 0

    def one_pass(r, _):
        base = (2 * r + c) * _RANGE
        nrows = jnp.minimum(_RANGE, _N_NODES - base)

        lax.fori_loop(0, _OUT_DIM // 16, zstage, 0)
        for q in range(-(-_RANGE // (16 * _NS))):
            cidx = t + _NS * q

            @pl.when(cidx * 16 < nrows)
            def _():
                pltpu.sync_copy(out_stage, acc.at[pl.ds(cidx * 16, 16)])

        @pl.when(t == _NS - 1)
        def _():
            pltpu.sync_copy(out_stage.at[pl.ds(0, 1)], acc.at[pl.ds(_RANGE, 1)])

        plsc.subcore_barrier()

        def cblock(b, cnt):
            da = pltpu.async_copy(
                dst_h.at[pl.ds(ebase + b * _DBLK, _DBLK)], dstblk, dma_sem)
            db = pltpu.async_copy(
                src_h.at[pl.ds(ebase + b * _DBLK, _DBLK)], srcblk, dma_sem)
            da.wait()
            db.wait()

            def cbody(i, cnt):
                d = dstblk[pl.ds(i * 16, 16)]
                s = srcblk[pl.ds(i * 16, 16)]
                m = (d >= base) & (d < base + _RANGE)
                pos = plsc.cumsum(jnp.where(m, 1, 0)) + cnt - 1
                packed = (b * _DBLK + i * 16 + iota) * 1024 + (d - base)
                plsc.store_scatter(ids_buf, [pos], packed, mask=m)
                plsc.store_scatter(srcs_buf, [pos], s, mask=m)
                return cnt + plsc.all_reduce_population_count(m)

            return lax.fori_loop(0, _DBLK // 16, cbody, cnt)

        cntv = lax.fori_loop(0, _EPT // _DBLK, cblock,
                             jnp.zeros((_LANES,), jnp.int32))
        plsc.store_scatter(ids_buf, [cntv + iota], _splat(0))
        plsc.store_scatter(srcs_buf, [cntv + iota], _splat(0))
        cnt = jnp.max(cntv)
        nch = (cnt + 15) // 16

        def chbody(ch, _):
            vv = ids_buf[pl.ds(ch * 16, 16)]
            validv = (ch * 16 + iota) < cntv
            ids_loc = lax.shift_right_logical(vv, 10)
            dstoff = vv & 1023
            eid_ib[...] = ids_loc + ebase
            dst_ib[...] = jnp.where(validv, dstoff, _RANGE)
            src_ib[...] = srcs_buf[pl.ds(ch * 16, 16)]
            dx = pltpu.async_copy(in1_h.at[src_ib], x_stage, dma_sem)
            dw = pltpu.async_copy(w_h.at[eid_ib], w_stage, dma_sem)
            dy = pltpu.async_copy(in2_h.at[eid_ib], y_stage, dma_sem)
            dx.wait()
            dw.wait()
            dy.wait()

            def xcopy(cb):
                for e in range(16):
                    x_buf[e, pl.ds(cb * 16, 16)] = x_stage[e, pl.ds(cb * 16, 16)]

            def wcopy(cb):
                for e in range(16):
                    w_buf[e, pl.ds(cb * 16, 16)] = w_stage[e, pl.ds(cb * 16, 16)]

            plsc.parallel_loop(0, _IN1_DIM // 16, 1)(xcopy)
            plsc.parallel_loop(0, _W_DIM // 16, 1)(wcopy)
            for e in range(16):
                y_buf[e, pl.ds(0, 16)] = y_stage[e, pl.ds(0, 16)]

            ys = [plsc.load_gather(y_buf, [iota, _splat(j)]) for j in range(_s2[-1])]

            for group in _GROUPS:
                st1 = 2 * group[0]['l1'] + 1
                xbase = group[0]['xbase']
                Ks = []
                for path in group:
                    K = {}
                    for (i, k), terms in path['knz'].items():
                        acc_v = None
                        for (j, coef) in terms:
                            term = ys[path['ybase'] + j] * coef
                            acc_v = term if acc_v is None else acc_v + term
                        K[(i, k)] = acc_v
                    Ks.append(K)
                i_used = sorted({i for K in Ks for (i, _k) in K})

                one_c = _splat(1)
                st1_c = _splat(st1)
                st3_cs = [_splat(2 * path['l3'] + 1) for path in group]
                carry0 = (
                    tuple(_splat(xbase + i) for i in i_used),
                    tuple(_splat(path['p'] * _MUL) for path in group),
                    tuple(_splat(path['obase']) for path in group),
                )

                def ubody(u, carry, group=group, Ks=Ks, i_used=i_used):
                    xcols, wcols, ocols = carry
                    xs = {
                        i: plsc.load_gather(x_buf, [iota, xcols[n]])
                        for n, i in enumerate(i_used)
                    }
                    for pi, (path, K) in enumerate(zip(group, Ks)):
                        st3 = 2 * path['l3'] + 1
                        wv = plsc.load_gather(w_buf, [iota, wcols[pi]])
                        idx = ocols[pi]
                        for k in range(st3):
                            acc_k = None
                            for i in i_used:
                                if (i, k) in K:
                                    term = K[(i, k)] * xs[i]
                                    acc_k = (term if acc_k is None
                                             else acc_k + term)
                            if acc_k is not None:
                                plsc.store_scatter(
                                    out_buf, [iota, idx], acc_k * wv)
                            if k + 1 < st3:
                                idx = idx + one_c
                    return (
                        tuple(xc + st1_c for xc in xcols),
                        tuple(wc + one_c for wc in wcols),
                        tuple(oc + st3_cs[pi] for pi, oc in enumerate(ocols)),
                    )

                plsc.parallel_loop(0, _MUL, 1, unroll=2, carry=carry0)(ubody)

            def ocopy(cb):
                for e in range(16):
                    out_stage[e, pl.ds(cb * 16, 16)] = out_buf[e, pl.ds(cb * 16, 16)]

            plsc.parallel_loop(0, _OUT_DIM // 16, 1)(ocopy)
            pltpu.sync_copy(out_stage, acc.at[dst_ib], add=True)
            return 0

        lax.fori_loop(0, nch, chbody, 0)
        plsc.subcore_barrier()

        for q in range(_RANGE // (16 * _NS)):
            cidx = t + _NS * q

            @pl.when(cidx * 16 < nrows)
            def _():
                pltpu.sync_copy(acc.at[pl.ds(cidx * 16, 16)],
                                out_h.at[pl.ds(base + cidx * 16, 16)])

        plsc.subcore_barrier()
        return 0

    lax.fori_loop(0, _NPASS, one_pass, 0)


def kernel(in1, in2, weight, per_edge_src, per_edge_dst):
    in2p = jnp.pad(in2, ((0, 0), (0, 16 - _s2[-1])))
    mesh = plsc.VectorSubcoreMesh(
        core_axis_name="c", subcore_axis_name="s",
        num_cores=_NC, num_subcores=_NS)
    f = pl.kernel(
        _body,
        out_type=jax.ShapeDtypeStruct((_N_NODES, _OUT_DIM), jnp.float32),
        mesh=mesh,
        compiler_params=pltpu.CompilerParams(
            use_tc_tiling_on_sc=False, needs_layout_passes=False),
        scratch_types=[
            pltpu.VMEM_SHARED((_RANGE + 1, _OUT_DIM), jnp.float32),
            pltpu.VMEM((_EPT + 16,), jnp.int32),
            pltpu.VMEM((_EPT + 16,), jnp.int32),
            pltpu.VMEM((_DBLK,), jnp.int32),
            pltpu.VMEM((_DBLK,), jnp.int32),
            pltpu.VMEM((16, _IN1_DIM), jnp.float32),
            pltpu.VMEM((16, _IN1_DIM + 1), jnp.float32),
            pltpu.VMEM((16, _W_DIM), jnp.float32),
            pltpu.VMEM((16, _W_DIM + 1), jnp.float32),
            pltpu.VMEM((16, 16), jnp.float32),
            pltpu.VMEM((16, 17), jnp.float32),
            pltpu.VMEM((16, _OUT_DIM + 1), jnp.float32),
            pltpu.VMEM((16, _OUT_DIM), jnp.float32),
            pltpu.VMEM((16,), jnp.int32),
            pltpu.VMEM((16,), jnp.int32),
            pltpu.VMEM((16,), jnp.int32),
            pltpu.SemaphoreType.DMA,
        ],
    )
    return f(in1, in2p, weight, per_edge_src, per_edge_dst)
